# baseline (speedup 1.0000x reference)
import functools

import jax
import jax.numpy as jnp
from jax import lax
from jax.experimental import pallas as pl
from jax.experimental.pallas import tpu as pltpu
from jax.experimental.pallas import tpu_sc as plsc

VOCAB_P1 = 100001
EMBED_DIM = 32
BATCH = 16384
_LANES = 16

_NUM_CORES = 2
_NUM_SUBCORES = 16
_CHUNK = BATCH // 4

_mesh = plsc.VectorSubcoreMesh(core_axis_name="c", subcore_axis_name="s")


@functools.partial(
    pl.kernel,
    mesh=_mesh,
    out_type=jax.ShapeDtypeStruct((EMBED_DIM, BATCH), jnp.float32),
    scratch_types=[
        pltpu.VMEM((VOCAB_P1,), jnp.float32),
        pltpu.VMEM((BATCH,), jnp.int32),
        pltpu.VMEM((_CHUNK,), jnp.float32),
        pltpu.VMEM((_CHUNK,), jnp.float32),
        pltpu.SemaphoreType.DMA,
        pltpu.SemaphoreType.DMA,
        pltpu.SemaphoreType.DMA,
    ],
    compiler_params=pltpu.CompilerParams(needs_layout_passes=False),
)
def _gather_feature_major(
    table_t_hbm, idx_hbm, out_t_hbm, row_v, idx_v, out0_v, out1_v,
    row_sem, idx_sem, out_sem,
):
    dim = lax.axis_index("c") * _NUM_SUBCORES + lax.axis_index("s")
    row_cp = pltpu.async_copy(table_t_hbm.at[dim], row_v, row_sem)
    idx_cp = pltpu.async_copy(idx_hbm, idx_v, idx_sem)
    idx_cp.wait()
    row_cp.wait()

    out_bufs = (out0_v, out1_v)
    pending = []
    for chunk in range(4):
        out_v = out_bufs[chunk % 2]
        if len(pending) >= 2:
            pending.pop(0).wait()

        def body(k, carry, chunk=chunk, out_v=out_v):
            base = chunk * _CHUNK + k * (8 * _LANES)
            ob = k * (8 * _LANES)
            for j in range(8):
                iv = idx_v[pl.ds(base + j * _LANES, _LANES)]
                out_v[pl.ds(ob + j * _LANES, _LANES)] = plsc.load_gather(row_v, [iv])
            return carry

        lax.fori_loop(0, _CHUNK // (8 * _LANES), body, 0)

        pending.append(pltpu.async_copy(
            out_v, out_t_hbm.at[dim, pl.ds(chunk * _CHUNK, _CHUNK)], out_sem))
    for cp in pending:
        cp.wait()


@jax.jit
def kernel(customer_id, user_embedding_table):
    out_t = _gather_feature_major(user_embedding_table.T, customer_id)
    return out_t.T


# speedup vs baseline: 1.1007x; 1.1007x over previous
import functools

import jax
import jax.numpy as jnp
from jax import lax
from jax.experimental import pallas as pl
from jax.experimental.pallas import tpu as pltpu
from jax.experimental.pallas import tpu_sc as plsc

VOCAB_P1 = 100001
EMBED_DIM = 32
BATCH = 16384
_LANES = 16

_NUM_CORES = 2
_NUM_SUBCORES = 16
_CHUNK = BATCH // 4

_mesh = plsc.VectorSubcoreMesh(core_axis_name="c", subcore_axis_name="s")


@functools.partial(
    pl.kernel,
    mesh=_mesh,
    out_type=jax.ShapeDtypeStruct((EMBED_DIM, BATCH), jnp.float32),
    scratch_types=[
        pltpu.VMEM((VOCAB_P1,), jnp.float32),
        pltpu.VMEM((BATCH,), jnp.int32),
        pltpu.VMEM((_CHUNK,), jnp.float32),
        pltpu.VMEM((_CHUNK,), jnp.float32),
        pltpu.SemaphoreType.DMA,
        pltpu.SemaphoreType.DMA,
        pltpu.SemaphoreType.DMA,
    ],
    compiler_params=pltpu.CompilerParams(needs_layout_passes=False),
)
def _gather_feature_major(
    table_t_hbm, idx_hbm, out_t_hbm, row_v, idx_v, out0_v, out1_v,
    row_sem, idx_sem, out_sem,
):
    dim = lax.axis_index("c") * _NUM_SUBCORES + lax.axis_index("s")
    row_cp = pltpu.async_copy(table_t_hbm.at[dim], row_v, row_sem)
    idx_cp = pltpu.async_copy(idx_hbm, idx_v, idx_sem)
    idx_cp.wait()
    row_cp.wait()

    out_bufs = (out0_v, out1_v)
    pending = []
    for chunk in range(4):
        out_v = out_bufs[chunk % 2]
        if len(pending) >= 2:
            pending.pop(0).wait()

        def body(k, carry, chunk=chunk, out_v=out_v):
            base = chunk * _CHUNK + k * (8 * _LANES)
            ob = k * (8 * _LANES)
            ivs = [idx_v[pl.ds(base + j * _LANES, _LANES)] for j in range(8)]
            vals = [plsc.load_gather(row_v, [iv]) for iv in ivs]
            for j in range(8):
                out_v[pl.ds(ob + j * _LANES, _LANES)] = vals[j]
            return carry

        lax.fori_loop(0, _CHUNK // (8 * _LANES), body, 0)

        pending.append(pltpu.async_copy(
            out_v, out_t_hbm.at[dim, pl.ds(chunk * _CHUNK, _CHUNK)], out_sem))
    for cp in pending:
        cp.wait()


@jax.jit
def kernel(customer_id, user_embedding_table):
    out_t = _gather_feature_major(user_embedding_table.T, customer_id)
    return out_t.T


# speedup vs baseline: 1.2387x; 1.1254x over previous
import functools

import jax
import jax.numpy as jnp
from jax import lax
from jax.experimental import pallas as pl
from jax.experimental.pallas import tpu as pltpu
from jax.experimental.pallas import tpu_sc as plsc

VOCAB_P1 = 100001
EMBED_DIM = 32
BATCH = 16384
_LANES = 16

_NUM_CORES = 2
_NUM_SUBCORES = 16
_CHUNK = BATCH // 4

_mesh = plsc.VectorSubcoreMesh(core_axis_name="c", subcore_axis_name="s")


@functools.partial(
    pl.kernel,
    mesh=_mesh,
    out_type=jax.ShapeDtypeStruct((EMBED_DIM, BATCH), jnp.float32),
    scratch_types=[
        pltpu.VMEM((VOCAB_P1,), jnp.float32),
        pltpu.VMEM((BATCH,), jnp.int32),
        pltpu.VMEM((_CHUNK,), jnp.float32),
        pltpu.VMEM((_CHUNK,), jnp.float32),
        pltpu.VMEM_SHARED((BATCH,), jnp.int32),
        pltpu.SemaphoreType.DMA,
        pltpu.SemaphoreType.DMA,
        pltpu.SemaphoreType.DMA,
    ],
    compiler_params=pltpu.CompilerParams(needs_layout_passes=False),
)
def _gather_feature_major(
    table_t_hbm, idx_hbm, out_t_hbm, row_v, idx_v, out0_v, out1_v,
    idx_shared, row_sem, idx_sem, out_sem,
):
    sid = lax.axis_index("s")
    dim = lax.axis_index("c") * _NUM_SUBCORES + sid
    row_cp = pltpu.async_copy(table_t_hbm.at[dim], row_v, row_sem)

    @pl.when(sid == 0)
    def _():
        pltpu.sync_copy(idx_hbm, idx_shared)

    plsc.subcore_barrier()
    idx_cp = pltpu.async_copy(idx_shared, idx_v, idx_sem)
    idx_cp.wait()
    row_cp.wait()

    out_bufs = (out0_v, out1_v)
    pending = []
    for chunk in range(4):
        out_v = out_bufs[chunk % 2]
        if len(pending) >= 2:
            pending.pop(0).wait()

        def body(k, carry, chunk=chunk, out_v=out_v):
            base = chunk * _CHUNK + k * (8 * _LANES)
            ob = k * (8 * _LANES)
            ivs = [idx_v[pl.ds(base + j * _LANES, _LANES)] for j in range(8)]
            vals = [plsc.load_gather(row_v, [iv]) for iv in ivs]
            for j in range(8):
                out_v[pl.ds(ob + j * _LANES, _LANES)] = vals[j]
            return carry

        lax.fori_loop(0, _CHUNK // (8 * _LANES), body, 0)

        pending.append(pltpu.async_copy(
            out_v, out_t_hbm.at[dim, pl.ds(chunk * _CHUNK, _CHUNK)], out_sem))
    for cp in pending:
        cp.wait()


@jax.jit
def kernel(customer_id, user_embedding_table):
    out_t = _gather_feature_major(user_embedding_table.T, customer_id)
    return out_t.T
